# baseline (speedup 1.0000x reference)
import functools

import jax
import jax.numpy as jnp
from jax import lax
from jax.experimental import pallas as pl
from jax.experimental.pallas import tpu as pltpu
from jax.experimental.pallas import tpu_sc as plsc

N = 4096
D = 128
E = 131072
DE = 16
NN = N * N

NC, NS = 2, 16
NW = NC * NS
EPW = E // NW
COLS = 128
ROWS = EPW // COLS
GRP = COLS // 16

BM = 256
EB = E // (N // BM)


def _tc_body(zi_ref, zall_ref, eat_ref, out_ref, en_ref):
    zz = lax.dot_general(
        zi_ref[...], zall_ref[...],
        (((1,), (1,)), ((), ())),
        preferred_element_type=jnp.float32,
    )
    out_ref[...] = 1.0 / (1.0 + jnp.exp(-zz))
    en_ref[...] = jnp.exp(jnp.sum(eat_ref[...], axis=0) * (-1.0 / DE))


def _dense_tiled(z, eat):
    return pl.pallas_call(
        _tc_body,
        grid=(N // BM,),
        in_specs=[
            pl.BlockSpec((BM, D), lambda i: (i, 0)),
            pl.BlockSpec((N, D), lambda i: (0, 0)),
            pl.BlockSpec((DE, EB), lambda i: (0, i)),
        ],
        out_specs=[
            pl.BlockSpec((BM, N), lambda i: (i, 0)),
            pl.BlockSpec((EB,), lambda i: (i,)),
        ],
        out_shape=[
            jax.ShapeDtypeStruct((N, N), jnp.float32),
            jax.ShapeDtypeStruct((E,), jnp.float32),
        ],
    )(z, z, eat)


def _tiled_words(dense):
    return dense.reshape(N // 8, 8, N // 128, 128).transpose(0, 2, 1, 3).reshape(NN)


def _untiled(flat):
    return flat.reshape(N // 8, N // 128, 8, 128).transpose(0, 2, 1, 3).reshape(N, N)


_mesh = plsc.VectorSubcoreMesh(
    core_axis_name="c", subcore_axis_name="s", num_cores=NC, num_subcores=NS)

NSEG = 16
SEG = NN // NSEG
SEGC = NSEG // NC
SLICE = SEG // NS
EPT = E // NS
CAP = 1024
SEGSH = 20


@functools.partial(
    pl.kernel,
    mesh=_mesh,
    compiler_params=pltpu.CompilerParams(needs_layout_passes=False),
    scratch_types=[
        pltpu.VMEM_SHARED((SEG + 16,), jnp.float32),
        pltpu.VMEM((EPT,), jnp.int32),
        pltpu.VMEM((EPT,), jnp.int32),
    ] + [pltpu.VMEM((CAP,), jnp.int32) for _ in range(SEGC)]
      + [
        pltpu.VMEM((SEGC, CAP), jnp.float32),
        pltpu.VMEM((CAP,), jnp.float32),
        pltpu.VMEM((EPT,), jnp.float32),
        pltpu.SemaphoreType.DMA,
    ],
)
def _sc_fix(out_hbm, ei_hbm, en_hbm, seg_v, r_v, c_v,
            b0, b1, b2, b3, b4, b5, b6, b7, ben_v, y_v, en_v, sem):
    bidxs = [b0, b1, b2, b3, b4, b5, b6, b7]
    cid = lax.axis_index("c")
    tid = lax.axis_index("s")
    ebase = tid * EPT
    pltpu.sync_copy(ei_hbm.at[0, pl.ds(ebase, EPT)], r_v)
    pltpu.sync_copy(ei_hbm.at[1, pl.ds(ebase, EPT)], c_v)
    pltpu.sync_copy(en_hbm.at[pl.ds(ebase, EPT)], en_v)

    dump = jnp.full((16,), SEG, jnp.int32)

    def pad(j, _):
        for b in range(SEGC):
            bidxs[b][pl.ds(j * 16, 16)] = dump
        return 0

    lax.fori_loop(0, CAP // 16, pad, 0)

    seg0 = cid * SEGC

    def bucket(j, cnts):
        r = r_v[pl.ds(j * 16, 16)]
        c = c_v[pl.ds(j * 16, 16)]
        idx = ((r >> 3) << 15) | ((c >> 7) << 10) | ((r & 7) << 7) | (c & 127)
        seg = idx >> SEGSH
        rel = idx & (SEG - 1)
        en = en_v[pl.ds(j * 16, 16)]
        new = []
        for b in range(SEGC):
            m = seg == (seg0 + b)
            cnt = cnts[b]
            plsc.store_compressed(bidxs[b].at[pl.ds(cnt, 16)], rel, mask=m)
            plsc.store_compressed(ben_v.at[b, pl.ds(cnt, 16)], en, mask=m)
            new.append(cnt + plsc.all_reduce_population_count(m)[0])
        return tuple(new)

    lax.fori_loop(0, EPT // 16, bucket, (jnp.int32(0),) * SEGC)

    for b in range(SEGC):
        sbase = (seg0 + b) * SEG
        pltpu.async_copy(out_hbm.at[pl.ds(sbase + tid * SLICE, SLICE)],
                         seg_v.at[pl.ds(tid * SLICE, SLICE)], sem).wait()
        plsc.subcore_barrier()
        pltpu.async_copy(seg_v.at[bidxs[b]], y_v, sem).wait()

        def f(j, _, _b=b):
            y = y_v[pl.ds(j * 16, 16)]
            en = ben_v[_b, pl.ds(j * 16, 16)]
            y_v[pl.ds(j * 16, 16)] = y / (y + en * (1.0 - y))
            return 0

        lax.fori_loop(0, CAP // 16, f, 0)
        pltpu.async_copy(y_v, seg_v.at[bidxs[b]], sem).wait()
        plsc.subcore_barrier()
        pltpu.async_copy(seg_v.at[pl.ds(tid * SLICE, SLICE)],
                         out_hbm.at[pl.ds(sbase + tid * SLICE, SLICE)],
                         sem).wait()
        plsc.subcore_barrier()


def kernel(z, edge_index, edge_attr):
    dense, en = _dense_tiled(z, edge_attr.T)
    ref = jax.new_ref(_tiled_words(dense))
    _sc_fix(ref, edge_index, en)
    return _untiled(ref[...])


# speedup vs baseline: 1.2471x; 1.2471x over previous
import functools

import jax
import jax.numpy as jnp
from jax import lax
from jax.experimental import pallas as pl
from jax.experimental.pallas import tpu as pltpu
from jax.experimental.pallas import tpu_sc as plsc

N = 4096
D = 128
E = 131072
DE = 16
NN = N * N

NC, NS = 2, 16
NW = NC * NS
EPW = E // NW
COLS = 128
ROWS = EPW // COLS
GRP = COLS // 16

BM = 256
EB = E // (N // BM)


def _tc_body(zi_ref, zall_ref, eat_ref, out_ref, en_ref):
    zz = lax.dot_general(
        zi_ref[...], zall_ref[...],
        (((1,), (1,)), ((), ())),
        preferred_element_type=jnp.float32,
    )
    out_ref[...] = 1.0 / (1.0 + jnp.exp(-zz))
    en_ref[...] = jnp.exp(jnp.sum(eat_ref[...], axis=0) * (-1.0 / DE))


def _dense_tiled(z, eat):
    return pl.pallas_call(
        _tc_body,
        grid=(N // BM,),
        in_specs=[
            pl.BlockSpec((BM, D), lambda i: (i, 0)),
            pl.BlockSpec((N, D), lambda i: (0, 0)),
            pl.BlockSpec((DE, EB), lambda i: (0, i)),
        ],
        out_specs=[
            pl.BlockSpec((BM, N), lambda i: (i, 0)),
            pl.BlockSpec((EB,), lambda i: (i,)),
        ],
        out_shape=[
            jax.ShapeDtypeStruct((N, N), jnp.float32),
            jax.ShapeDtypeStruct((E,), jnp.float32),
        ],
    )(z, z, eat)


def _tiled_words(dense):
    return dense.reshape(N // 8, 8, N // 128, 128).transpose(0, 2, 1, 3).reshape(NN)


def _untiled(flat):
    return flat.reshape(N // 8, N // 128, 8, 128).transpose(0, 2, 1, 3).reshape(N, N)


_mesh = plsc.VectorSubcoreMesh(
    core_axis_name="c", subcore_axis_name="s", num_cores=NC, num_subcores=NS)


NQ = 4
CH = EPW // NQ


@functools.partial(
    pl.kernel,
    mesh=_mesh,
    compiler_params=pltpu.CompilerParams(needs_layout_passes=False),
    scratch_types=[
        pltpu.VMEM((EPW,), jnp.int32),
        pltpu.VMEM((EPW,), jnp.int32),
        pltpu.VMEM((EPW,), jnp.float32),
    ] + [pltpu.VMEM((CH,), jnp.int32) for _ in range(NQ)]
      + [pltpu.VMEM((CH,), jnp.float32) for _ in range(NQ)]
      + [pltpu.SemaphoreType.DMA for _ in range(NQ)]
      + [
        pltpu.SemaphoreType.DMA,
        pltpu.SemaphoreType.DMA,
    ],
)
def _sc_fix(out_hbm, ei_hbm, en_hbm, r_v, c_v, en_v,
            i0, i1, i2, i3, y0, y1, y2, y3, g0, g1, g2, g3, lsem, ssem):
    idxs = [i0, i1, i2, i3]
    ys = [y0, y1, y2, y3]
    gsems = [g0, g1, g2, g3]
    wid = lax.axis_index("s") * NC + lax.axis_index("c")
    base = wid * EPW
    pltpu.sync_copy(ei_hbm.at[0, pl.ds(base, EPW)], r_v)
    pltpu.sync_copy(ei_hbm.at[1, pl.ds(base, EPW)], c_v)
    en_load = pltpu.async_copy(en_hbm.at[pl.ds(base, EPW)], en_v, lsem)

    gathers = []
    for q in range(NQ):
        def build(j, _, _q=q):
            for k in range(GRP):
                off = _q * CH + j * COLS + k * 16
                r = r_v[pl.ds(off, 16)]
                c = c_v[pl.ds(off, 16)]
                idxs[_q][pl.ds(j * COLS + k * 16, 16)] = (
                    ((r >> 3) << 15) | ((c >> 7) << 10)
                    | ((r & 7) << 7) | (c & 127)
                )
            return 0

        lax.fori_loop(0, CH // COLS, build, 0)
        gathers.append(pltpu.async_copy(out_hbm.at[idxs[q]], ys[q], gsems[q]))

    en_load.wait()
    scatters = []
    for q in range(NQ):
        gathers[q].wait()

        def fix(j, _, _q=q):
            y = ys[_q][pl.ds(j * 16, 16)]
            en = en_v[pl.ds(_q * CH + j * 16, 16)]
            ys[_q][pl.ds(j * 16, 16)] = y / (y + en * (1.0 - y))
            return 0

        lax.fori_loop(0, CH // 16, fix, 0)
        scatters.append(pltpu.async_copy(ys[q], out_hbm.at[idxs[q]], ssem))
    for sc in scatters:
        sc.wait()


def kernel(z, edge_index, edge_attr):
    dense, en = _dense_tiled(z, edge_attr.T)
    ref = jax.new_ref(_tiled_words(dense))
    _sc_fix(ref, edge_index, en)
    return _untiled(ref[...])


# speedup vs baseline: 1.2765x; 1.0236x over previous
import functools

import jax
import jax.numpy as jnp
from jax import lax
from jax.experimental import pallas as pl
from jax.experimental.pallas import tpu as pltpu
from jax.experimental.pallas import tpu_sc as plsc

N = 4096
D = 128
E = 131072
DE = 16
NN = N * N

NC, NS = 2, 16
NW = NC * NS
EPW = E // NW
COLS = 128
ROWS = EPW // COLS
GRP = COLS // 16

BM = 512
EB = E // (N // BM)


def _tc_body(zi_ref, zall_ref, eat_ref, out_ref, en_ref):
    zz = lax.dot_general(
        zi_ref[...], zall_ref[...],
        (((1,), (1,)), ((), ())),
        preferred_element_type=jnp.float32,
    )
    out_ref[...] = 1.0 / (1.0 + jnp.exp(-zz))
    en_ref[...] = jnp.exp(jnp.sum(eat_ref[...], axis=0) * (-1.0 / DE))


def _dense_tiled(z, eat):
    return pl.pallas_call(
        _tc_body,
        grid=(N // BM,),
        in_specs=[
            pl.BlockSpec((BM, D), lambda i: (i, 0)),
            pl.BlockSpec((N, D), lambda i: (0, 0)),
            pl.BlockSpec((DE, EB), lambda i: (0, i)),
        ],
        out_specs=[
            pl.BlockSpec((BM, N), lambda i: (i, 0)),
            pl.BlockSpec((EB,), lambda i: (i,)),
        ],
        out_shape=[
            jax.ShapeDtypeStruct((N, N), jnp.float32),
            jax.ShapeDtypeStruct((E,), jnp.float32),
        ],
    )(z, z, eat)


def _tiled_words(dense):
    return dense.reshape(N // 8, 8, N // 128, 128).transpose(0, 2, 1, 3).reshape(NN)


def _untiled(flat):
    return flat.reshape(N // 8, N // 128, 8, 128).transpose(0, 2, 1, 3).reshape(N, N)


_mesh = plsc.VectorSubcoreMesh(
    core_axis_name="c", subcore_axis_name="s", num_cores=NC, num_subcores=NS)


NQ = 4
CH = EPW // NQ


@functools.partial(
    pl.kernel,
    mesh=_mesh,
    compiler_params=pltpu.CompilerParams(needs_layout_passes=False),
    scratch_types=[
        pltpu.VMEM((EPW,), jnp.int32),
        pltpu.VMEM((EPW,), jnp.int32),
        pltpu.VMEM((EPW,), jnp.float32),
    ] + [pltpu.VMEM((CH,), jnp.int32) for _ in range(NQ)]
      + [pltpu.VMEM((CH,), jnp.float32) for _ in range(NQ)]
      + [pltpu.SemaphoreType.DMA for _ in range(NQ)]
      + [
        pltpu.SemaphoreType.DMA,
        pltpu.SemaphoreType.DMA,
    ],
)
def _sc_fix(out_hbm, ei_hbm, en_hbm, r_v, c_v, en_v,
            i0, i1, i2, i3, y0, y1, y2, y3, g0, g1, g2, g3, lsem, ssem):
    idxs = [i0, i1, i2, i3]
    ys = [y0, y1, y2, y3]
    gsems = [g0, g1, g2, g3]
    wid = lax.axis_index("s") * NC + lax.axis_index("c")
    base = wid * EPW
    pltpu.sync_copy(ei_hbm.at[0, pl.ds(base, EPW)], r_v)
    pltpu.sync_copy(ei_hbm.at[1, pl.ds(base, EPW)], c_v)
    en_load = pltpu.async_copy(en_hbm.at[pl.ds(base, EPW)], en_v, lsem)

    gathers = []
    for q in range(NQ):
        def build(j, _, _q=q):
            for k in range(GRP):
                off = _q * CH + j * COLS + k * 16
                r = r_v[pl.ds(off, 16)]
                c = c_v[pl.ds(off, 16)]
                idxs[_q][pl.ds(j * COLS + k * 16, 16)] = (
                    ((r >> 3) << 15) | ((c >> 7) << 10)
                    | ((r & 7) << 7) | (c & 127)
                )
            return 0

        lax.fori_loop(0, CH // COLS, build, 0)
        gathers.append(pltpu.async_copy(out_hbm.at[idxs[q]], ys[q], gsems[q]))

    en_load.wait()
    scatters = []
    for q in range(NQ):
        gathers[q].wait()

        def fix(j, _, _q=q):
            y = ys[_q][pl.ds(j * 16, 16)]
            en = en_v[pl.ds(_q * CH + j * 16, 16)]
            ys[_q][pl.ds(j * 16, 16)] = y / (y + en * (1.0 - y))
            return 0

        lax.fori_loop(0, CH // 16, fix, 0)
        scatters.append(pltpu.async_copy(ys[q], out_hbm.at[idxs[q]], ssem))
    for sc in scatters:
        sc.wait()


def kernel(z, edge_index, edge_attr):
    dense, en = _dense_tiled(z, edge_attr.T)
    ref = jax.new_ref(_tiled_words(dense))
    _sc_fix(ref, edge_index, en)
    return _untiled(ref[...])
